# baseline (speedup 1.0000x reference)
import functools

import jax
import jax.numpy as jnp
from jax import lax
from jax.experimental import pallas as pl
from jax.experimental.pallas import tpu as pltpu
from jax.experimental.pallas import tpu_sc as plsc

NUM_EMBED = 1000000
EMBED_DIM = 64
OUTPUT_DIM = 128
BATCH = 16384
FIELDS = 26
ROWS = BATCH * FIELDS

NC = 2
NS = 16
NW = NC * NS
K = 4
RCHUNK = ROWS // K
RPW = RCHUNK // NW
CH = 128
NCH = RPW // CH
GRP = CH // 16

CVT_BO = 2048
CVT_GRID = (NUM_EMBED + CVT_BO - 1) // CVT_BO


def _cvt_body(a_ref, eye_ref, o_ref):
    o_ref[...] = jax.lax.dot_general(
        a_ref[...], eye_ref[...],
        dimension_numbers=(((0,), (0,)), ((), ())),
        preferred_element_type=jnp.float32,
    )


def _convert(tab_t, eye):
    return pl.pallas_call(
        _cvt_body,
        grid=(CVT_GRID,),
        in_specs=[
            pl.BlockSpec((EMBED_DIM, CVT_BO), lambda i: (0, i)),
            pl.BlockSpec((EMBED_DIM, EMBED_DIM), lambda i: (0, 0)),
        ],
        out_specs=pl.BlockSpec((CVT_BO, EMBED_DIM), lambda i: (i, 0)),
        out_shape=jax.ShapeDtypeStruct((NUM_EMBED, EMBED_DIM), jnp.float32),
    )(tab_t, eye)


def _gather_body(idx_hbm, tab_hbm, h_hbm, idx_v, rows_v, gsem, ssem):
    wid = lax.axis_index("s") * NC + lax.axis_index("c")
    hbase = wid * RPW
    pltpu.sync_copy(idx_hbm.at[wid], idx_v)

    def fire_burst(c, slot):
        lanes = lax.iota(jnp.int32, 16)
        for g in range(GRP):
            vec = idx_v[pl.ds(c * CH + g * 16, 16)]
            for j in range(16):
                r = jnp.sum(vec * (lanes == j).astype(jnp.int32))
                pltpu.make_async_copy(
                    tab_hbm.at[pl.ds(r, 1)],
                    rows_v.at[slot, pl.ds(g * 16 + j, 1)],
                    gsem.at[slot],
                ).start()

    def wait_burst(slot):
        pltpu.make_async_copy(
            tab_hbm.at[pl.ds(0, CH)],
            rows_v.at[slot],
            gsem.at[slot],
        ).wait()

    def fire_store(c, slot):
        pltpu.make_async_copy(
            rows_v.at[slot],
            h_hbm.at[pl.ds(hbase + c * CH, CH)],
            ssem.at[slot],
        ).start()

    def wait_store(slot):
        pltpu.make_async_copy(
            rows_v.at[slot],
            h_hbm.at[pl.ds(hbase, CH)],
            ssem.at[slot],
        ).wait()

    fire_burst(0, 0)

    def step(c, carry):
        slot = c % 2

        @pl.when(c + 1 < NCH)
        def _():
            @pl.when(c >= 1)
            def _():
                wait_store(1 - slot)

            fire_burst(c + 1, 1 - slot)

        wait_burst(slot)
        fire_store(c, slot)
        return carry

    lax.fori_loop(0, NCH, step, 0)
    wait_store(0)
    wait_store(1)


@functools.cache
def _make_gather():
    return pl.kernel(
        _gather_body,
        mesh=plsc.VectorSubcoreMesh(core_axis_name="c", subcore_axis_name="s"),
        out_type=jax.ShapeDtypeStruct((RCHUNK, EMBED_DIM), jnp.float32),
        compiler_params=pltpu.CompilerParams(needs_layout_passes=False),
        scratch_types=[
            pltpu.VMEM((RPW,), jnp.int32),
            pltpu.VMEM((2, CH, EMBED_DIM), jnp.float32),
            pltpu.SemaphoreType.DMA((2,)),
            pltpu.SemaphoreType.DMA((2,)),
        ],
    )


MM_BLK = 2048
MM_GRID = RCHUNK // MM_BLK


def _mm_body0(h_ref, wt_ref, b_ref, o_ref):
    o_ref[...] = (
        jnp.dot(h_ref[...], wt_ref[...], preferred_element_type=jnp.float32)
        + b_ref[...]
    )


def _mm_bodyk(h_ref, wt_ref, b_ref, carry_ref, o_ref):
    del carry_ref
    o_ref[...] = (
        jnp.dot(h_ref[...], wt_ref[...], preferred_element_type=jnp.float32)
        + b_ref[...]
    )


def _matmul_chunk(h2k, w2, b2, k, out2_prev):
    if k == 0:
        return pl.pallas_call(
            _mm_body0,
            grid=(MM_GRID,),
            in_specs=[
                pl.BlockSpec((MM_BLK, EMBED_DIM), lambda i: (i, 0)),
                pl.BlockSpec((EMBED_DIM, OUTPUT_DIM), lambda i: (0, 0)),
                pl.BlockSpec((1, OUTPUT_DIM), lambda i: (0, 0)),
            ],
            out_specs=pl.BlockSpec((MM_BLK, OUTPUT_DIM), lambda i: (i, 0)),
            out_shape=jax.ShapeDtypeStruct((ROWS, OUTPUT_DIM), jnp.float32),
        )(h2k, w2, b2)
    return pl.pallas_call(
        _mm_bodyk,
        grid=(MM_GRID,),
        in_specs=[
            pl.BlockSpec((MM_BLK, EMBED_DIM), lambda i: (i, 0)),
            pl.BlockSpec((EMBED_DIM, OUTPUT_DIM), lambda i: (0, 0)),
            pl.BlockSpec((1, OUTPUT_DIM), lambda i: (0, 0)),
            pl.BlockSpec(memory_space=pl.ANY),
        ],
        out_specs=pl.BlockSpec((MM_BLK, OUTPUT_DIM),
                               lambda i, k=k: (k * MM_GRID + i, 0)),
        out_shape=jax.ShapeDtypeStruct((ROWS, OUTPUT_DIM), jnp.float32),
        input_output_aliases={3: 0},
    )(h2k, w2, b2, out2_prev)


def kernel(x, table, W, b):
    idx = x.T.reshape(ROWS).astype(jnp.int32)
    tab = _convert(table.T, jnp.eye(EMBED_DIM, dtype=jnp.float32))
    wt = W.T
    b2 = b.reshape(1, OUTPUT_DIM)
    gather = _make_gather()
    out2 = None
    for k in range(K):
        idx_k = lax.slice(idx, (k * RCHUNK,), ((k + 1) * RCHUNK,)).reshape(
            NW, RPW)
        h2k = gather(idx_k, tab)
        out2 = _matmul_chunk(h2k, wt, b2, k, out2)
    return out2.reshape(FIELDS, BATCH, OUTPUT_DIM).transpose(1, 0, 2)


# speedup vs baseline: 1.2003x; 1.2003x over previous
import functools

import jax
import jax.numpy as jnp
from jax import lax
from jax.experimental import pallas as pl
from jax.experimental.pallas import tpu as pltpu
from jax.experimental.pallas import tpu_sc as plsc

NUM_EMBED = 1000000
EMBED_DIM = 64
OUTPUT_DIM = 128
BATCH = 16384
FIELDS = 26
ROWS = BATCH * FIELDS
NC = 2
NS = 16
NW = NC * NS
K = 2
RCHUNK = ROWS // K
RPW = RCHUNK // NW
CH = 128
NCH = RPW // CH
GRP = CH // 16

def _gather_body(idx_hbm, tab_hbm, h2_hbm, idx_v, rows_v, gsem, ssem):
    wid = lax.axis_index("s") * NC + lax.axis_index("c")
    hbase = wid * RPW
    pltpu.sync_copy(idx_hbm.at[wid], idx_v)

    def fire_burst(c, slot):
        lanes = lax.iota(jnp.int32, 16)
        for g in range(GRP):
            vec = idx_v[pl.ds(c * CH + g * 16, 16)]
            for j in range(16):
                r = jnp.sum(vec * (lanes == j).astype(jnp.int32))
                pltpu.make_async_copy(
                    tab_hbm.at[pl.ds(r, 1)],
                    rows_v.at[slot, pl.ds(g * 16 + j, 1)],
                    gsem.at[slot],
                ).start()

    def wait_burst(slot):
        pltpu.make_async_copy(
            tab_hbm.at[pl.ds(0, CH)],
            rows_v.at[slot],
            gsem.at[slot],
        ).wait()

    def fire_store(c, slot):
        pltpu.make_async_copy(
            rows_v.at[slot],
            h2_hbm.at[pl.ds(hbase + c * CH, CH)],
            ssem.at[slot],
        ).start()

    def wait_store(slot):
        pltpu.make_async_copy(
            rows_v.at[slot],
            h2_hbm.at[pl.ds(hbase, CH)],
            ssem.at[slot],
        ).wait()

    fire_burst(0, 0)

    def step(c, carry):
        slot = c % 2

        @pl.when(c + 1 < NCH)
        def _():
            @pl.when(c >= 1)
            def _():
                wait_store(1 - slot)

            fire_burst(c + 1, 1 - slot)

        wait_burst(slot)
        fire_store(c, slot)
        return carry

    lax.fori_loop(0, NCH, step, 0)
    wait_store(0)
    wait_store(1)


@functools.cache
def _make_gather():
    return pl.kernel(
        _gather_body,
        mesh=plsc.VectorSubcoreMesh(core_axis_name="c", subcore_axis_name="s"),
        out_type=jax.ShapeDtypeStruct((RCHUNK, EMBED_DIM), jnp.float32),
        compiler_params=pltpu.CompilerParams(needs_layout_passes=False),
        scratch_types=[
            pltpu.VMEM((RPW,), jnp.int32),
            pltpu.VMEM((2, CH, EMBED_DIM), jnp.float32),
            pltpu.SemaphoreType.DMA((2,)),
            pltpu.SemaphoreType.DMA((2,)),
        ],
    )


MM_BLK = 2048
MM_GRID = RCHUNK // MM_BLK


def _mm_body0(h_ref, wt_ref, b_ref, o_ref):
    o_ref[...] = (
        jnp.dot(h_ref[...], wt_ref[...], preferred_element_type=jnp.float32)
        + b_ref[...]
    )


def _mm_bodyk(h_ref, wt_ref, b_ref, carry_ref, o_ref):
    del carry_ref
    o_ref[...] = (
        jnp.dot(h_ref[...], wt_ref[...], preferred_element_type=jnp.float32)
        + b_ref[...]
    )


def _matmul_chunk(h2k, w2, b2, k, out2_prev):
    if k == 0:
        return pl.pallas_call(
            _mm_body0,
            grid=(MM_GRID,),
            in_specs=[
                pl.BlockSpec((MM_BLK, EMBED_DIM), lambda i: (i, 0)),
                pl.BlockSpec((EMBED_DIM, OUTPUT_DIM), lambda i: (0, 0)),
                pl.BlockSpec((1, OUTPUT_DIM), lambda i: (0, 0)),
            ],
            out_specs=pl.BlockSpec((MM_BLK, OUTPUT_DIM), lambda i: (i, 0)),
            out_shape=jax.ShapeDtypeStruct((ROWS, OUTPUT_DIM), jnp.float32),
        )(h2k, w2, b2)
    return pl.pallas_call(
        _mm_bodyk,
        grid=(MM_GRID,),
        in_specs=[
            pl.BlockSpec((MM_BLK, EMBED_DIM), lambda i: (i, 0)),
            pl.BlockSpec((EMBED_DIM, OUTPUT_DIM), lambda i: (0, 0)),
            pl.BlockSpec((1, OUTPUT_DIM), lambda i: (0, 0)),
            pl.BlockSpec(memory_space=pl.ANY),
        ],
        out_specs=pl.BlockSpec((MM_BLK, OUTPUT_DIM),
                               lambda i, k=k: (k * MM_GRID + i, 0)),
        out_shape=jax.ShapeDtypeStruct((ROWS, OUTPUT_DIM), jnp.float32),
        input_output_aliases={3: 0},
    )(h2k, w2, b2, out2_prev)


def kernel(x, table, W, b):
    idx = x.T.reshape(ROWS).astype(jnp.int32)
    wt = W.T
    b2 = b.reshape(1, OUTPUT_DIM)
    gather = _make_gather()
    out2 = None
    for k in range(K):
        idx_k = lax.slice(idx, (k * RCHUNK,), ((k + 1) * RCHUNK,)).reshape(
            NW, RPW)
        h2k = gather(idx_k, table)
        out2 = _matmul_chunk(h2k, wt, b2, k, out2)
    return out2.reshape(FIELDS, BATCH, OUTPUT_DIM).transpose(1, 0, 2)


# speedup vs baseline: 1.2867x; 1.0720x over previous
import functools

import jax
import jax.numpy as jnp
from jax import lax
from jax.experimental import pallas as pl
from jax.experimental.pallas import tpu as pltpu
from jax.experimental.pallas import tpu_sc as plsc

NUM_EMBED = 1000000
EMBED_DIM = 64
OUTPUT_DIM = 128
BATCH = 16384
FIELDS = 26
ROWS = BATCH * FIELDS

NC = 2
NS = 16
NW = NC * NS
RPW = ROWS // NW
CH = 128
NCH = RPW // CH
GRP = CH // 16

PBLK = 2048
PGRID = (NUM_EMBED + PBLK - 1) // PBLK


def _proj_body(t_ref, w_ref, b_ref, o_ref):
    o_ref[...] = jax.lax.dot_general(
        t_ref[...], w_ref[...],
        dimension_numbers=(((0,), (0,)), ((), ())),
        preferred_element_type=jnp.float32,
    ) + b_ref[...]


def _project(tab_t, wt, b2):
    return pl.pallas_call(
        _proj_body,
        grid=(PGRID,),
        in_specs=[
            pl.BlockSpec((EMBED_DIM, PBLK), lambda i: (0, i)),
            pl.BlockSpec((EMBED_DIM, OUTPUT_DIM), lambda i: (0, 0)),
            pl.BlockSpec((1, OUTPUT_DIM), lambda i: (0, 0)),
        ],
        out_specs=pl.BlockSpec((PBLK, OUTPUT_DIM), lambda i: (i, 0)),
        out_shape=jax.ShapeDtypeStruct((NUM_EMBED, OUTPUT_DIM), jnp.float32),
    )(tab_t, wt, b2)


def _gather_body(idx_hbm, p_hbm, out_hbm, idx_v, rows_v, gsem, ssem):
    wid = lax.axis_index("s") * NC + lax.axis_index("c")
    obase = wid * RPW
    pltpu.sync_copy(idx_hbm.at[wid], idx_v)

    def fire_burst(c, slot):
        lanes = lax.iota(jnp.int32, 16)
        for g in range(GRP):
            vec = idx_v[pl.ds(c * CH + g * 16, 16)]
            for j in range(16):
                r = jnp.sum(vec * (lanes == j).astype(jnp.int32))
                pltpu.make_async_copy(
                    p_hbm.at[pl.ds(r, 1)],
                    rows_v.at[slot, pl.ds(g * 16 + j, 1)],
                    gsem.at[slot],
                ).start()

    def wait_burst(slot):
        pltpu.make_async_copy(
            p_hbm.at[pl.ds(0, CH)],
            rows_v.at[slot],
            gsem.at[slot],
        ).wait()

    def fire_store(c, slot):
        pltpu.make_async_copy(
            rows_v.at[slot],
            out_hbm.at[pl.ds(obase + c * CH, CH)],
            ssem.at[slot],
        ).start()

    def wait_store(slot):
        pltpu.make_async_copy(
            rows_v.at[slot],
            out_hbm.at[pl.ds(obase, CH)],
            ssem.at[slot],
        ).wait()

    fire_burst(0, 0)

    def step(c, carry):
        slot = c % 2

        @pl.when(c + 1 < NCH)
        def _():
            @pl.when(c >= 1)
            def _():
                wait_store(1 - slot)

            fire_burst(c + 1, 1 - slot)

        wait_burst(slot)
        fire_store(c, slot)
        return carry

    lax.fori_loop(0, NCH, step, 0)
    wait_store(0)
    wait_store(1)


@functools.cache
def _make_gather():
    return pl.kernel(
        _gather_body,
        mesh=plsc.VectorSubcoreMesh(core_axis_name="c", subcore_axis_name="s"),
        out_type=jax.ShapeDtypeStruct((ROWS, OUTPUT_DIM), jnp.float32),
        compiler_params=pltpu.CompilerParams(needs_layout_passes=False),
        scratch_types=[
            pltpu.VMEM((RPW,), jnp.int32),
            pltpu.VMEM((2, CH, OUTPUT_DIM), jnp.float32),
            pltpu.SemaphoreType.DMA((2,)),
            pltpu.SemaphoreType.DMA((2,)),
        ],
    )


def kernel(x, table, W, b):
    idx = x.T.reshape(NW, RPW).astype(jnp.int32)
    p = _project(table.T, W.T, b.reshape(1, OUTPUT_DIM))
    out2 = _make_gather()(idx, p)
    return out2.reshape(FIELDS, BATCH, OUTPUT_DIM).transpose(1, 0, 2)


# speedup vs baseline: 1.6396x; 1.2742x over previous
import functools

import jax
import jax.numpy as jnp
from jax import lax
from jax.experimental import pallas as pl
from jax.experimental.pallas import tpu as pltpu
from jax.experimental.pallas import tpu_sc as plsc

NUM_EMBED = 1000000
EMBED_DIM = 64
OUTPUT_DIM = 128
BATCH = 16384
FIELDS = 26
ROWS = BATCH * FIELDS

NC = 2
NS = 16
NW = NC * NS
RPW = ROWS // NW
CH = 128
NCH = RPW // CH
GRP = CH // 16

PBLK = 4096
PGRID = (NUM_EMBED + PBLK - 1) // PBLK


def _proj_body(t_ref, w_ref, b_ref, o_ref):
    o_ref[...] = jax.lax.dot_general(
        t_ref[...], w_ref[...],
        dimension_numbers=(((0,), (0,)), ((), ())),
        preferred_element_type=jnp.float32,
    ) + b_ref[...]


def _project(tab_t, wt, b2):
    return pl.pallas_call(
        _proj_body,
        grid=(PGRID,),
        in_specs=[
            pl.BlockSpec((EMBED_DIM, PBLK), lambda i: (0, i)),
            pl.BlockSpec((EMBED_DIM, OUTPUT_DIM), lambda i: (0, 0)),
            pl.BlockSpec((1, OUTPUT_DIM), lambda i: (0, 0)),
        ],
        out_specs=pl.BlockSpec((PBLK, OUTPUT_DIM), lambda i: (i, 0)),
        out_shape=jax.ShapeDtypeStruct((NUM_EMBED, OUTPUT_DIM), jnp.float32),
    )(tab_t, wt, b2)


def _gather_body(idx_hbm, p_hbm, out_hbm, idx_v, rows_v, gsem, ssem):
    wid = lax.axis_index("s") * NC + lax.axis_index("c")
    obase = wid * RPW
    pltpu.sync_copy(idx_hbm.at[wid], idx_v)

    def fire_burst(c, slot):
        lanes = lax.iota(jnp.int32, 16)
        for g in range(GRP):
            vec = idx_v[pl.ds(c * CH + g * 16, 16)]
            for j in range(16):
                r = jnp.sum(vec * (lanes == j).astype(jnp.int32))
                pltpu.make_async_copy(
                    p_hbm.at[pl.ds(r, 1)],
                    rows_v.at[slot, pl.ds(g * 16 + j, 1)],
                    gsem.at[slot],
                ).start()

    def wait_burst(slot):
        pltpu.make_async_copy(
            p_hbm.at[pl.ds(0, CH)],
            rows_v.at[slot],
            gsem.at[slot],
        ).wait()

    def fire_store(c, slot):
        pltpu.make_async_copy(
            rows_v.at[slot],
            out_hbm.at[pl.ds(obase + c * CH, CH)],
            ssem.at[slot],
        ).start()

    def wait_store(slot):
        pltpu.make_async_copy(
            rows_v.at[slot],
            out_hbm.at[pl.ds(obase, CH)],
            ssem.at[slot],
        ).wait()

    fire_burst(0, 0)

    def step(c, carry):
        slot = c % 2

        @pl.when(c + 1 < NCH)
        def _():
            @pl.when(c >= 1)
            def _():
                wait_store(1 - slot)

            fire_burst(c + 1, 1 - slot)

        wait_burst(slot)
        fire_store(c, slot)
        return carry

    lax.fori_loop(0, NCH, step, 0)
    wait_store(0)
    wait_store(1)


@functools.cache
def _make_gather():
    return pl.kernel(
        _gather_body,
        mesh=plsc.VectorSubcoreMesh(core_axis_name="c", subcore_axis_name="s"),
        out_type=jax.ShapeDtypeStruct((ROWS, OUTPUT_DIM), jnp.float32),
        compiler_params=pltpu.CompilerParams(needs_layout_passes=False),
        scratch_types=[
            pltpu.VMEM((RPW,), jnp.int32),
            pltpu.VMEM((2, CH, OUTPUT_DIM), jnp.float32),
            pltpu.SemaphoreType.DMA((2,)),
            pltpu.SemaphoreType.DMA((2,)),
        ],
    )


def kernel(x, table, W, b):
    idx = x.T.reshape(NW, RPW).astype(jnp.int32)
    p = _project(table.T, W.T, b.reshape(1, OUTPUT_DIM))
    out2 = _make_gather()(idx, p)
    return out2.reshape(FIELDS, BATCH, OUTPUT_DIM).transpose(1, 0, 2)


# speedup vs baseline: 1.9131x; 1.1668x over previous
import functools

import jax
import jax.numpy as jnp
from jax import lax
from jax.experimental import pallas as pl
from jax.experimental.pallas import tpu as pltpu
from jax.experimental.pallas import tpu_sc as plsc

NUM_EMBED = 1000000
EMBED_DIM = 64
OUTPUT_DIM = 128
BATCH = 16384
FIELDS = 26
ROWS = BATCH * FIELDS

NC = 2
NS = 16
NW = NC * NS
RPW = ROWS // NW
CH = 128
NCH = RPW // CH
GRP = CH // 16

PBLK = 8192
PGRID = (NUM_EMBED + PBLK - 1) // PBLK


def _proj_body(t_ref, w_ref, b_ref, o_ref):
    o_ref[...] = jax.lax.dot_general(
        t_ref[...], w_ref[...],
        dimension_numbers=(((0,), (0,)), ((), ())),
        preferred_element_type=jnp.float32,
    ) + b_ref[...]


def _project(tab_t, wt, b2):
    return pl.pallas_call(
        _proj_body,
        grid=(PGRID,),
        in_specs=[
            pl.BlockSpec((EMBED_DIM, PBLK), lambda i: (0, i)),
            pl.BlockSpec((EMBED_DIM, OUTPUT_DIM), lambda i: (0, 0)),
            pl.BlockSpec((1, OUTPUT_DIM), lambda i: (0, 0)),
        ],
        out_specs=pl.BlockSpec((PBLK, OUTPUT_DIM), lambda i: (i, 0)),
        out_shape=jax.ShapeDtypeStruct((NUM_EMBED, OUTPUT_DIM), jnp.float32),
    )(tab_t, wt, b2)


def _gather_body(idx_hbm, p_hbm, out_hbm, idx_v, rows_v, gsem, ssem):
    wid = lax.axis_index("s") * NC + lax.axis_index("c")
    obase = wid * RPW
    pltpu.sync_copy(idx_hbm.at[wid], idx_v)

    def fire_burst(c, slot):
        lanes = lax.iota(jnp.int32, 16)
        for g in range(GRP):
            vec = idx_v[pl.ds(c * CH + g * 16, 16)]
            for j in range(16):
                r = jnp.sum(vec * (lanes == j).astype(jnp.int32))
                pltpu.make_async_copy(
                    p_hbm.at[pl.ds(r, 1)],
                    rows_v.at[slot, pl.ds(g * 16 + j, 1)],
                    gsem.at[slot],
                ).start()

    def wait_burst(slot):
        pltpu.make_async_copy(
            p_hbm.at[pl.ds(0, CH)],
            rows_v.at[slot],
            gsem.at[slot],
        ).wait()

    def fire_store(c, slot):
        pltpu.make_async_copy(
            rows_v.at[slot],
            out_hbm.at[pl.ds(obase + c * CH, CH)],
            ssem.at[slot],
        ).start()

    def wait_store(slot):
        pltpu.make_async_copy(
            rows_v.at[slot],
            out_hbm.at[pl.ds(obase, CH)],
            ssem.at[slot],
        ).wait()

    fire_burst(0, 0)

    def step(c, carry):
        slot = c % 2

        @pl.when(c + 1 < NCH)
        def _():
            @pl.when(c >= 1)
            def _():
                wait_store(1 - slot)

            fire_burst(c + 1, 1 - slot)

        wait_burst(slot)
        fire_store(c, slot)
        return carry

    lax.fori_loop(0, NCH, step, 0)
    wait_store(0)
    wait_store(1)


@functools.cache
def _make_gather():
    return pl.kernel(
        _gather_body,
        mesh=plsc.VectorSubcoreMesh(core_axis_name="c", subcore_axis_name="s"),
        out_type=jax.ShapeDtypeStruct((ROWS, OUTPUT_DIM), jnp.float32),
        compiler_params=pltpu.CompilerParams(needs_layout_passes=False),
        scratch_types=[
            pltpu.VMEM((RPW,), jnp.int32),
            pltpu.VMEM((2, CH, OUTPUT_DIM), jnp.float32),
            pltpu.SemaphoreType.DMA((2,)),
            pltpu.SemaphoreType.DMA((2,)),
        ],
    )


def kernel(x, table, W, b):
    idx = x.T.reshape(NW, RPW).astype(jnp.int32)
    p = _project(table.T, W.T, b.reshape(1, OUTPUT_DIM))
    out2 = _make_gather()(idx, p)
    return out2.reshape(FIELDS, BATCH, OUTPUT_DIM).transpose(1, 0, 2)


# speedup vs baseline: 2.0267x; 1.0594x over previous
import functools

import jax
import jax.numpy as jnp
from jax import lax
from jax.experimental import pallas as pl
from jax.experimental.pallas import tpu as pltpu
from jax.experimental.pallas import tpu_sc as plsc

NUM_EMBED = 1000000
EMBED_DIM = 64
OUTPUT_DIM = 128
BATCH = 16384
FIELDS = 26
ROWS = BATCH * FIELDS

NC = 2
NS = 16
NW = NC * NS
RPW = ROWS // NW
CH = 128
NCH = RPW // CH
GRP = CH // 16

PBLK = 16384
PGRID = (NUM_EMBED + PBLK - 1) // PBLK


def _proj_body(t_ref, w_ref, b_ref, o_ref):
    o_ref[...] = jax.lax.dot_general(
        t_ref[...], w_ref[...],
        dimension_numbers=(((0,), (0,)), ((), ())),
        preferred_element_type=jnp.float32,
    ) + b_ref[...]


def _project(tab_t, wt, b2):
    return pl.pallas_call(
        _proj_body,
        grid=(PGRID,),
        in_specs=[
            pl.BlockSpec((EMBED_DIM, PBLK), lambda i: (0, i)),
            pl.BlockSpec((EMBED_DIM, OUTPUT_DIM), lambda i: (0, 0)),
            pl.BlockSpec((1, OUTPUT_DIM), lambda i: (0, 0)),
        ],
        out_specs=pl.BlockSpec((PBLK, OUTPUT_DIM), lambda i: (i, 0)),
        out_shape=jax.ShapeDtypeStruct((NUM_EMBED, OUTPUT_DIM), jnp.float32),
    )(tab_t, wt, b2)


def _gather_body(idx_hbm, p_hbm, out_hbm, idx_v, rows_v, gsem, ssem):
    wid = lax.axis_index("s") * NC + lax.axis_index("c")
    obase = wid * RPW
    pltpu.sync_copy(idx_hbm.at[wid], idx_v)

    def fire_burst(c, slot):
        lanes = lax.iota(jnp.int32, 16)
        for g in range(GRP):
            vec = idx_v[pl.ds(c * CH + g * 16, 16)]
            for j in range(16):
                r = jnp.sum(vec * (lanes == j).astype(jnp.int32))
                pltpu.make_async_copy(
                    p_hbm.at[pl.ds(r, 1)],
                    rows_v.at[slot, pl.ds(g * 16 + j, 1)],
                    gsem.at[slot],
                ).start()

    def wait_burst(slot):
        pltpu.make_async_copy(
            p_hbm.at[pl.ds(0, CH)],
            rows_v.at[slot],
            gsem.at[slot],
        ).wait()

    def fire_store(c, slot):
        pltpu.make_async_copy(
            rows_v.at[slot],
            out_hbm.at[pl.ds(obase + c * CH, CH)],
            ssem.at[slot],
        ).start()

    def wait_store(slot):
        pltpu.make_async_copy(
            rows_v.at[slot],
            out_hbm.at[pl.ds(obase, CH)],
            ssem.at[slot],
        ).wait()

    fire_burst(0, 0)

    def step(c, carry):
        slot = c % 2

        @pl.when(c + 1 < NCH)
        def _():
            @pl.when(c >= 1)
            def _():
                wait_store(1 - slot)

            fire_burst(c + 1, 1 - slot)

        wait_burst(slot)
        fire_store(c, slot)
        return carry

    lax.fori_loop(0, NCH, step, 0)
    wait_store(0)
    wait_store(1)


@functools.cache
def _make_gather():
    return pl.kernel(
        _gather_body,
        mesh=plsc.VectorSubcoreMesh(core_axis_name="c", subcore_axis_name="s"),
        out_type=jax.ShapeDtypeStruct((ROWS, OUTPUT_DIM), jnp.float32),
        compiler_params=pltpu.CompilerParams(needs_layout_passes=False),
        scratch_types=[
            pltpu.VMEM((RPW,), jnp.int32),
            pltpu.VMEM((2, CH, OUTPUT_DIM), jnp.float32),
            pltpu.SemaphoreType.DMA((2,)),
            pltpu.SemaphoreType.DMA((2,)),
        ],
    )


def kernel(x, table, W, b):
    idx = x.T.reshape(NW, RPW).astype(jnp.int32)
    p = _project(table.T, W.T, b.reshape(1, OUTPUT_DIM))
    out2 = _make_gather()(idx, p)
    return out2.reshape(FIELDS, BATCH, OUTPUT_DIM).transpose(1, 0, 2)


# speedup vs baseline: 2.0669x; 1.0198x over previous
import functools

import jax
import jax.numpy as jnp
from jax import lax
from jax.experimental import pallas as pl
from jax.experimental.pallas import tpu as pltpu
from jax.experimental.pallas import tpu_sc as plsc

NUM_EMBED = 1000000
EMBED_DIM = 64
OUTPUT_DIM = 128
BATCH = 16384
FIELDS = 26
ROWS = BATCH * FIELDS

NC = 2
NS = 16
NW = NC * NS
RPW = ROWS // NW
CH = 128
NCH = RPW // CH
GRP = CH // 16

PBLK = 32768
PGRID = (NUM_EMBED + PBLK - 1) // PBLK


def _proj_body(t_ref, w_ref, b_ref, o_ref):
    o_ref[...] = jax.lax.dot_general(
        t_ref[...], w_ref[...],
        dimension_numbers=(((0,), (0,)), ((), ())),
        preferred_element_type=jnp.float32,
    ) + b_ref[...]


def _project(tab_t, wt, b2):
    return pl.pallas_call(
        _proj_body,
        grid=(PGRID,),
        in_specs=[
            pl.BlockSpec((EMBED_DIM, PBLK), lambda i: (0, i)),
            pl.BlockSpec((EMBED_DIM, OUTPUT_DIM), lambda i: (0, 0)),
            pl.BlockSpec((1, OUTPUT_DIM), lambda i: (0, 0)),
        ],
        out_specs=pl.BlockSpec((PBLK, OUTPUT_DIM), lambda i: (i, 0)),
        out_shape=jax.ShapeDtypeStruct((NUM_EMBED, OUTPUT_DIM), jnp.float32),
    )(tab_t, wt, b2)


def _gather_body(idx_hbm, p_hbm, out_hbm, idx_v, rows_v, gsem, ssem):
    wid = lax.axis_index("s") * NC + lax.axis_index("c")
    obase = wid * RPW
    pltpu.sync_copy(idx_hbm.at[wid], idx_v)

    def fire_burst(c, slot):
        lanes = lax.iota(jnp.int32, 16)
        for g in range(GRP):
            vec = idx_v[pl.ds(c * CH + g * 16, 16)]
            for j in range(16):
                r = jnp.sum(vec * (lanes == j).astype(jnp.int32))
                pltpu.make_async_copy(
                    p_hbm.at[pl.ds(r, 1)],
                    rows_v.at[slot, pl.ds(g * 16 + j, 1)],
                    gsem.at[slot],
                ).start()

    def wait_burst(slot):
        pltpu.make_async_copy(
            p_hbm.at[pl.ds(0, CH)],
            rows_v.at[slot],
            gsem.at[slot],
        ).wait()

    def fire_store(c, slot):
        pltpu.make_async_copy(
            rows_v.at[slot],
            out_hbm.at[pl.ds(obase + c * CH, CH)],
            ssem.at[slot],
        ).start()

    def wait_store(slot):
        pltpu.make_async_copy(
            rows_v.at[slot],
            out_hbm.at[pl.ds(obase, CH)],
            ssem.at[slot],
        ).wait()

    fire_burst(0, 0)

    def step(c, carry):
        slot = c % 2

        @pl.when(c + 1 < NCH)
        def _():
            @pl.when(c >= 1)
            def _():
                wait_store(1 - slot)

            fire_burst(c + 1, 1 - slot)

        wait_burst(slot)
        fire_store(c, slot)
        return carry

    lax.fori_loop(0, NCH, step, 0)
    wait_store(0)
    wait_store(1)


@functools.cache
def _make_gather():
    return pl.kernel(
        _gather_body,
        mesh=plsc.VectorSubcoreMesh(core_axis_name="c", subcore_axis_name="s"),
        out_type=jax.ShapeDtypeStruct((ROWS, OUTPUT_DIM), jnp.float32),
        compiler_params=pltpu.CompilerParams(needs_layout_passes=False),
        scratch_types=[
            pltpu.VMEM((RPW,), jnp.int32),
            pltpu.VMEM((2, CH, OUTPUT_DIM), jnp.float32),
            pltpu.SemaphoreType.DMA((2,)),
            pltpu.SemaphoreType.DMA((2,)),
        ],
    )


def kernel(x, table, W, b):
    idx = x.T.reshape(NW, RPW).astype(jnp.int32)
    p = _project(table.T, W.T, b.reshape(1, OUTPUT_DIM))
    out2 = _make_gather()(idx, p)
    return out2.reshape(FIELDS, BATCH, OUTPUT_DIM).transpose(1, 0, 2)
